# Initial kernel scaffold; baseline (speedup 1.0000x reference)
import jax
import jax.numpy as jnp
from jax.experimental import pallas as pl


def kernel(x, ln_gamma, ln_beta, a_weight, b_weight):
    raise NotImplementedError("write your pallas kernel here")


# speedup vs baseline: 16.2842x; 16.2842x over previous
import functools

import jax
import jax.numpy as jnp
from jax import lax
from jax.experimental import pallas as pl
from jax.experimental.pallas import tpu as pltpu
from jax.experimental.pallas import tpu_sc as plsc

IN_FEATURES = 256
OUT_FEATURES = 64
GRID_SIZE = 16
GRID_MIN = -1.0
INV_STEP = 8.0
BATCH = 1024
LANES = 16
NWORKERS = 32
BPW = BATCH // NWORKERS
CHUNK = 128
NCHUNK = IN_FEATURES // CHUNK


def _splat(s, dtype=None):
    v = lax.broadcast(s, (LANES,))
    return v if dtype is None else v.astype(dtype)


_GDN = lax.GatherDimensionNumbers(
    offset_dims=(), collapsed_slice_dims=(0,), start_index_map=(0,))


def _lane_perm(v, idx):
    return lax.gather(v, idx[:, None], _GDN, slice_sizes=(1,),
                      mode=lax.GatherScatterMode.PROMISE_IN_BOUNDS)


def _lane_allsum(v):
    lane = lax.iota(jnp.int32, LANES)
    for sh in (8, 4, 2, 1):
        v = v + _lane_perm(v, lax.bitwise_xor(lane, sh))
    return v


@functools.partial(
    pl.kernel,
    out_type=jax.ShapeDtypeStruct((BATCH, OUT_FEATURES), jnp.float32),
    mesh=plsc.VectorSubcoreMesh(core_axis_name="c", subcore_axis_name="s"),
    compiler_params=pltpu.CompilerParams(needs_layout_passes=False),
    scratch_types=[
        pltpu.VMEM((BPW, IN_FEATURES), jnp.float32),
        pltpu.VMEM((BPW, IN_FEATURES), jnp.int32),
        pltpu.VMEM((2, CHUNK, 128), jnp.float32),
        pltpu.VMEM((BPW, OUT_FEATURES), jnp.float32),
        pltpu.VMEM((IN_FEATURES,), jnp.float32),
        pltpu.VMEM((IN_FEATURES,), jnp.float32),
        pltpu.SemaphoreType.DMA,
        pltpu.SemaphoreType.DMA,
    ],
)
def _sc_kernel(x_hbm, gam_hbm, bet_hbm, w_hbm, out_hbm,
               xn_v, idx_v, rows_v, acc_v, gam_v, bet_v, sem0, sem1):
    wid = lax.axis_index("s") * 2 + lax.axis_index("c")
    base = wid * BPW

    pltpu.sync_copy(x_hbm.at[pl.ds(base, BPW)], xn_v)
    pltpu.sync_copy(gam_hbm, gam_v)
    pltpu.sync_copy(bet_hbm, bet_v)

    zero16 = jnp.zeros((LANES,), jnp.float32)

    def ln_row(b, carry):
        def red(k, sc):
            s, ss = sc
            v = xn_v[b, pl.ds(k * LANES, LANES)]
            return s + v, ss + v * v

        s, ss = lax.fori_loop(0, IN_FEATURES // LANES, red, (zero16, zero16))
        mean_v = _lane_allsum(s) * (1.0 / IN_FEATURES)
        var_v = _lane_allsum(ss) * (1.0 / IN_FEATURES) - mean_v * mean_v
        tv = var_v + 1e-5
        iv = plsc.bitcast(tv, jnp.int32)
        y = plsc.bitcast(jnp.int32(0x5F3759DF) - (iv >> 1), jnp.float32)
        y = y * (1.5 - 0.5 * tv * y * y)
        y = y * (1.5 - 0.5 * tv * y * y)
        y = y * (1.5 - 0.5 * tv * y * y)
        lane = lax.iota(jnp.int32, LANES)

        def norm(k, c):
            sl = pl.ds(k * LANES, LANES)
            xv = xn_v[b, sl]
            xn = (xv - mean_v) * y * gam_v[sl] + bet_v[sl]
            fi = (xn - GRID_MIN) * INV_STEP
            seg = jnp.clip(fi.astype(jnp.int32), 0, GRID_SIZE - 1)
            xn_v[b, sl] = xn
            idx_v[b, sl] = seg + (k * LANES + lane) * GRID_SIZE
            return c

        return lax.fori_loop(0, IN_FEATURES // LANES, norm, carry)

    lax.fori_loop(0, BPW, ln_row, 0)

    sems = (sem0, sem1)

    def fire(c, p):
        pltpu.async_copy(
            w_hbm.at[idx_v.at[c // NCHUNK, pl.ds((c % NCHUNK) * CHUNK, CHUNK)]],
            rows_v.at[p], sems[p])

    def drain(c, p):
        pltpu.make_async_copy(
            w_hbm.at[idx_v.at[c // NCHUNK, pl.ds((c % NCHUNK) * CHUNK, CHUNK)]],
            rows_v.at[p], sems[p]).wait()

    fire(0, 0)

    def chunk_pair(cp, carry):
        accs = carry

        def one(c, p, accs):
            b = c // NCHUNK
            fired = c + 1 < BPW * NCHUNK
            lax.cond(fired, lambda: fire(c + 1, 1 - p), lambda: None)
            drain(c, p)

            def feat(i, a):
                xs = plsc.load_gather(
                    xn_v, [_splat(b), _splat((c % NCHUNK) * CHUNK + i)])
                a0, a1, a2, a3 = a
                r = rows_v
                a0 = a0 + xs * r[p, i, pl.ds(0, 16)] + r[p, i, pl.ds(64, 16)]
                a1 = a1 + xs * r[p, i, pl.ds(16, 16)] + r[p, i, pl.ds(80, 16)]
                a2 = a2 + xs * r[p, i, pl.ds(32, 16)] + r[p, i, pl.ds(96, 16)]
                a3 = a3 + xs * r[p, i, pl.ds(48, 16)] + r[p, i, pl.ds(112, 16)]
                return (a0, a1, a2, a3)

            accs = lax.fori_loop(0, CHUNK, feat, accs)

            def flush():
                acc_v[b, pl.ds(0, 16)] = accs[0]
                acc_v[b, pl.ds(16, 16)] = accs[1]
                acc_v[b, pl.ds(32, 16)] = accs[2]
                acc_v[b, pl.ds(48, 16)] = accs[3]

            is_last = (c % NCHUNK) == (NCHUNK - 1)
            lax.cond(is_last, flush, lambda: None)
            keep = lax.broadcast(is_last, (LANES,))
            accs = tuple(lax.select(keep, zero16, a) for a in accs)
            return accs

        accs = one(2 * cp, 0, accs)
        accs = one(2 * cp + 1, 1, accs)
        return accs

    lax.fori_loop(0, BPW * NCHUNK // 2, chunk_pair,
                  (zero16, zero16, zero16, zero16))

    pltpu.sync_copy(acc_v, out_hbm.at[pl.ds(base, BPW)])


def kernel(x, ln_gamma, ln_beta, a_weight, b_weight):
    w_cat = jnp.concatenate([a_weight, b_weight], axis=1)
    return _sc_kernel(x, ln_gamma, ln_beta, w_cat)
